# Initial kernel scaffold; baseline (speedup 1.0000x reference)
import jax
import jax.numpy as jnp
from jax.experimental import pallas as pl


def kernel(A, guide):
    raise NotImplementedError("write your pallas kernel here")


# speedup vs baseline: 6958.2330x; 6958.2330x over previous
import functools

import jax
import jax.numpy as jnp
from jax.experimental import pallas as pl


BH = 64
CQ = 3
NQ = 12 // CQ


def _upsample_body(a_ref, v_ref, t_ref):
    t_ref[0] = jnp.dot(a_ref[0], v_ref[...], preferred_element_type=jnp.float32)


def _slice_body(t_ref, u_ref, g_ref, o_ref):
    cmat = jnp.dot(u_ref[...], t_ref[0], preferred_element_type=jnp.float32)
    g = g_ref[0]
    tz = jnp.clip((g + 1.0) * 3.5, 0.0, 7.0)
    wzs = [jnp.maximum(0.0, 1.0 - jnp.abs(tz - z)) for z in range(8)]
    for cc in range(CQ):
        acc = wzs[0] * cmat[:, (cc * 8) * 512:(cc * 8 + 1) * 512]
        for z in range(1, 8):
            acc = acc + wzs[z] * cmat[:, (cc * 8 + z) * 512:(cc * 8 + z + 1) * 512]
        o_ref[0, :, 0, cc, :] = acc


def _tent_matrix(npix, ngrid):
    gx = jnp.linspace(-1.0, 1.0, npix, dtype=jnp.float32)
    t = jnp.clip((gx + 1.0) * 0.5 * (ngrid - 1), 0.0, float(ngrid - 1))
    i = jnp.arange(ngrid, dtype=jnp.float32)
    return jnp.maximum(0.0, 1.0 - jnp.abs(t[:, None] - i[None, :]))


@jax.jit
def kernel(A, guide):
    bs, H, W, _ = guide.shape
    C = A.shape[1]
    g1, g2, g3 = A.shape[2], A.shape[3], A.shape[4]

    U = _tent_matrix(H, g1)
    VT = U.T

    A5 = jnp.transpose(A, (0, 2, 1, 4, 3)).reshape(bs, g1 * C * g3, g2)

    T0 = pl.pallas_call(
        _upsample_body,
        grid=(bs,),
        in_specs=[
            pl.BlockSpec((1, g1 * C * g3, g2), lambda b: (b, 0, 0)),
            pl.BlockSpec((g2, W), lambda b: (0, 0)),
        ],
        out_specs=pl.BlockSpec((1, g1 * C * g3, W), lambda b: (b, 0, 0)),
        out_shape=jax.ShapeDtypeStruct((bs, g1 * C * g3, W), jnp.float32),
    )(A5, VT)

    T = T0.reshape(bs, g1, C * g3 * W)
    guide2 = guide.reshape(bs, H, W)

    O = pl.pallas_call(
        _slice_body,
        grid=(bs, NQ, H // BH),
        in_specs=[
            pl.BlockSpec((1, g1, CQ * g3 * W), lambda b, q, h: (b, 0, q)),
            pl.BlockSpec((BH, g1), lambda b, q, h: (h, 0)),
            pl.BlockSpec((1, BH, W), lambda b, q, h: (b, h, 0)),
        ],
        out_specs=pl.BlockSpec((1, BH, 1, CQ, W), lambda b, q, h: (b, h, q, 0, 0)),
        out_shape=jax.ShapeDtypeStruct((bs, H, NQ, CQ, W), jnp.float32),
    )(T, U, guide2)

    return jnp.transpose(O.reshape(bs, H, C, W), (0, 1, 3, 2))
